# Initial kernel scaffold; baseline (speedup 1.0000x reference)
import jax
import jax.numpy as jnp
from jax.experimental import pallas as pl


def kernel(x, edge_index, counts, out_edge_idx, layer_i, W, att_i, att_j):
    raise NotImplementedError("write your pallas kernel here")


# speedup vs baseline: 8.9279x; 8.9279x over previous
import functools

import jax
import jax.numpy as jnp
from jax import lax
from jax.experimental import pallas as pl
from jax.experimental.pallas import tpu as pltpu
from jax.experimental.pallas import tpu_sc as plsc

N = 10000
E = 320000
HF = 128
HA = 144
NH = 4
NC, NS = 2, 16
NW = NC * NS
EPT = E // NW
CHUNK = 80
NCH = EPT // CHUNK
NCHP = 128

_mesh = plsc.VectorSubcoreMesh(
    core_axis_name="c", subcore_axis_name="s", num_cores=NC, num_subcores=NS)

ROWS_BLK = 1000


def _tc_front(x_ref, wt_ref, ai_ref, aj_ref, h_ref, sa_ref, sb_ref,
              ma_ref, mb_ref):
    i = pl.program_id(0)
    h = jnp.dot(x_ref[...], wt_ref[...], preferred_element_type=jnp.float32)
    h_ref[...] = jnp.concatenate(
        [h, jnp.zeros((ROWS_BLK, HA - HF), jnp.float32)], axis=1)
    sa = jnp.dot(h, ai_ref[...], preferred_element_type=jnp.float32)
    sb = jnp.dot(h, aj_ref[...], preferred_element_type=jnp.float32)
    sa_ref[...] = sa
    sb_ref[...] = sb

    @pl.when(i == 0)
    def _():
        ma_ref[...] = jnp.full((8, 16), -jnp.inf, jnp.float32)
        mb_ref[...] = jnp.full((8, 16), -jnp.inf, jnp.float32)

    ma_ref[...] = jnp.maximum(ma_ref[...], jnp.max(sa, axis=0)[None, :])
    mb_ref[...] = jnp.maximum(mb_ref[...], jnp.max(sb, axis=0)[None, :])


def _run_tc_front(x, wt, amat_i, amat_j):
    return pl.pallas_call(
        _tc_front,
        grid=(N // ROWS_BLK,),
        in_specs=[
            pl.BlockSpec((ROWS_BLK, 128), lambda i: (i, 0)),
            pl.BlockSpec((128, 128), lambda i: (0, 0)),
            pl.BlockSpec((128, 16), lambda i: (0, 0)),
            pl.BlockSpec((128, 16), lambda i: (0, 0)),
        ],
        out_specs=[
            pl.BlockSpec((ROWS_BLK, HA), lambda i: (i, 0)),
            pl.BlockSpec((ROWS_BLK, 16), lambda i: (i, 0)),
            pl.BlockSpec((ROWS_BLK, 16), lambda i: (i, 0)),
            pl.BlockSpec((8, 16), lambda i: (0, 0)),
            pl.BlockSpec((8, 16), lambda i: (0, 0)),
        ],
        out_shape=[
            jax.ShapeDtypeStruct((N, HA), jnp.float32),
            jax.ShapeDtypeStruct((N, 16), jnp.float32),
            jax.ShapeDtypeStruct((N, 16), jnp.float32),
            jax.ShapeDtypeStruct((8, 16), jnp.float32),
            jax.ShapeDtypeStruct((8, 16), jnp.float32),
        ],
    )(x, wt, amat_i, amat_j)


@functools.partial(
    pl.kernel,
    out_type=jax.ShapeDtypeStruct((NC * N, HA), jnp.float32),
    mesh=_mesh,
    compiler_params=pltpu.CompilerParams(
        needs_layout_passes=False, use_tc_tiling_on_sc=False),
    scratch_types=[
        pltpu.VMEM((CHUNK,), jnp.int32),
        pltpu.VMEM((CHUNK,), jnp.int32),
        pltpu.VMEM((CHUNK, 16), jnp.float32),
        pltpu.VMEM((CHUNK, 16), jnp.float32),
        pltpu.VMEM((CHUNK, HA), jnp.float32),
        pltpu.VMEM((16,), jnp.float32),
        pltpu.VMEM((16,), jnp.float32),
        pltpu.VMEM_SHARED((N, HA), jnp.float32),
        pltpu.SemaphoreType.DMA,
    ],
)
def _sc_main(h_hbm, sa_hbm, sb_hbm, src_hbm, dst_hbm, ma_hbm, mb_hbm, z_hbm,
             acc_hbm,
             src_c, dst_c, sd_v, ss_v, rows_v, ma_v, mb_v, acc_sh, sem):
    cid = lax.axis_index("c")
    tid = lax.axis_index("s")
    wid = cid * NS + tid

    pltpu.sync_copy(ma_hbm.at[pl.ds(0, 16)], ma_v)
    pltpu.sync_copy(mb_hbm.at[pl.ds(0, 16)], mb_v)

    @pl.when(tid < NS - 1)
    def _():
        pltpu.sync_copy(z_hbm.at[pl.ds(tid * 632, 632)],
                        acc_sh.at[pl.ds(tid * 632, 632)])

    @pl.when(tid == NS - 1)
    def _():
        pltpu.sync_copy(z_hbm.at[pl.ds(9480, 520)],
                        acc_sh.at[pl.ds(9480, 520)])

    plsc.subcore_barrier()

    m0 = ma_v[...] + mb_v[...]
    m4 = jnp.maximum(m0, 0.01 * m0)
    dmask = jnp.where(lax.iota(jnp.int32, 16) < NH, 1.0, 0.0
                      ).astype(jnp.float32)

    def chunk_body(c, carry):
        row = wid * NCHP + c
        pltpu.sync_copy(src_hbm.at[row], src_c)
        pltpu.sync_copy(dst_hbm.at[row], dst_c)
        g1 = pltpu.async_copy(h_hbm.at[src_c], rows_v, sem)
        g2 = pltpu.async_copy(sa_hbm.at[dst_c], sd_v, sem)
        g3 = pltpu.async_copy(sb_hbm.at[src_c], ss_v, sem)
        g1.wait()
        g2.wait()
        g3.wait()

        def edge_body(e, ecarry):
            z = sd_v[e, pl.ds(0, 16)] + ss_v[e, pl.ds(0, 16)]
            l = jnp.maximum(z, 0.01 * z)
            p16 = jnp.exp(l - m4)
            for hd in range(NH):
                cv = jnp.full((16,), p16[hd], jnp.float32)
                lo = rows_v[e, pl.ds(hd * 32, 16)] * cv
                rows_v[e, pl.ds(hd * 32, 16)] = lo
                hi = rows_v[e, pl.ds(hd * 32 + 16, 16)] * cv
                rows_v[e, pl.ds(hd * 32 + 16, 16)] = hi
            rows_v[e, pl.ds(HF, 16)] = p16 * dmask
            return ecarry

        lax.fori_loop(0, CHUNK, edge_body, 0)
        pltpu.sync_copy(rows_v, acc_sh.at[dst_c], add=True)
        return carry

    lax.fori_loop(0, NCH, chunk_body, 0)
    plsc.subcore_barrier()

    @pl.when(tid < NS - 1)
    def _():
        pltpu.sync_copy(acc_sh.at[pl.ds(tid * 632, 632)],
                        acc_hbm.at[pl.ds(cid * N + tid * 632, 632)])

    @pl.when(tid == NS - 1)
    def _():
        pltpu.sync_copy(acc_sh.at[pl.ds(9480, 520)],
                        acc_hbm.at[pl.ds(cid * N + 9480, 520)])


def _tc_combine(a0_ref, a1_ref, b_ref, o_ref):
    a = a0_ref[...] + a1_ref[...]
    d = a[:, HF:HF + NH]
    r = 1.0 / (d + 1e-16)
    o_ref[...] = a[:, :HF] * jnp.dot(
        r, b_ref[...], preferred_element_type=jnp.float32)


def _run_tc_combine(acc, bmat):
    nb = N // ROWS_BLK
    return pl.pallas_call(
        _tc_combine,
        grid=(nb,),
        in_specs=[
            pl.BlockSpec((ROWS_BLK, HA), lambda i: (i, 0)),
            pl.BlockSpec((ROWS_BLK, HA), lambda i: (nb + i, 0)),
            pl.BlockSpec((NH, HF), lambda i: (0, 0)),
        ],
        out_specs=pl.BlockSpec((ROWS_BLK, HF), lambda i: (i, 0)),
        out_shape=jax.ShapeDtypeStruct((N, HF), jnp.float32),
    )(acc, acc, bmat)


def kernel(x, edge_index, counts, out_edge_idx, layer_i, W, att_i, att_j):
    ei = edge_index.astype(jnp.int32).reshape(2, NW, NCH, CHUNK)
    ei = jnp.pad(ei, ((0, 0), (0, 0), (0, NCHP - NCH), (0, 0)))
    src = ei[0].reshape(NW * NCHP, CHUNK)
    dst = ei[1].reshape(NW * NCHP, CHUNK)
    wt = W.T.astype(jnp.float32)
    bmat = jnp.repeat(jnp.eye(NH, dtype=jnp.float32), HF // NH, axis=1)
    pad12 = jnp.zeros((HF, 12), jnp.float32)
    amat_i = jnp.concatenate([bmat.T * att_i.reshape(-1)[:, None], pad12], 1)
    amat_j = jnp.concatenate([bmat.T * att_j.reshape(-1)[:, None], pad12], 1)
    h, sa, sb, ma, mb = _run_tc_front(x, wt, amat_i, amat_j)
    z = jnp.zeros((N, HA), jnp.float32)
    acc = _sc_main(h, sa, sb, src, dst, ma.reshape(-1), mb.reshape(-1), z)
    return _run_tc_combine(acc, bmat)


# speedup vs baseline: 16.3107x; 1.8269x over previous
import functools

import jax
import jax.numpy as jnp
from jax import lax
from jax.experimental import pallas as pl
from jax.experimental.pallas import tpu as pltpu
from jax.experimental.pallas import tpu_sc as plsc

N = 10000
E = 320000
HF = 128
HA = 144
NH = 4
NC, NS = 2, 16
NW = NC * NS
EPT = E // NW
CHUNK = 80
NCH = EPT // CHUNK
NCHP = 128

_mesh = plsc.VectorSubcoreMesh(
    core_axis_name="c", subcore_axis_name="s", num_cores=NC, num_subcores=NS)

ROWS_BLK = 1000


def _tc_front(x_ref, wt_ref, ai_ref, aj_ref, h_ref, sa_ref, ma_ref, mb_ref):
    i = pl.program_id(0)
    h = jnp.dot(x_ref[...], wt_ref[...], preferred_element_type=jnp.float32)
    sa = jnp.dot(h, ai_ref[...], preferred_element_type=jnp.float32)
    sb = jnp.dot(h, aj_ref[...], preferred_element_type=jnp.float32)
    h_ref[...] = jnp.concatenate([h, sb], axis=1)
    sa_ref[...] = sa

    @pl.when(i == 0)
    def _():
        ma_ref[...] = jnp.full((8, 16), -jnp.inf, jnp.float32)
        mb_ref[...] = jnp.full((8, 16), -jnp.inf, jnp.float32)

    ma_ref[...] = jnp.maximum(ma_ref[...], jnp.max(sa, axis=0)[None, :])
    mb_ref[...] = jnp.maximum(mb_ref[...], jnp.max(sb, axis=0)[None, :])


def _run_tc_front(x, wt, amat_i, amat_j):
    return pl.pallas_call(
        _tc_front,
        grid=(N // ROWS_BLK,),
        in_specs=[
            pl.BlockSpec((ROWS_BLK, 128), lambda i: (i, 0)),
            pl.BlockSpec((128, 128), lambda i: (0, 0)),
            pl.BlockSpec((128, 16), lambda i: (0, 0)),
            pl.BlockSpec((128, 16), lambda i: (0, 0)),
        ],
        out_specs=[
            pl.BlockSpec((ROWS_BLK, HA), lambda i: (i, 0)),
            pl.BlockSpec((ROWS_BLK, 16), lambda i: (i, 0)),
            pl.BlockSpec((8, 16), lambda i: (0, 0)),
            pl.BlockSpec((8, 16), lambda i: (0, 0)),
        ],
        out_shape=[
            jax.ShapeDtypeStruct((N, HA), jnp.float32),
            jax.ShapeDtypeStruct((N, 16), jnp.float32),
            jax.ShapeDtypeStruct((8, 16), jnp.float32),
            jax.ShapeDtypeStruct((8, 16), jnp.float32),
        ],
    )(x, wt, amat_i, amat_j)


@functools.partial(
    pl.kernel,
    out_type=jax.ShapeDtypeStruct((NC * N, HA), jnp.float32),
    mesh=_mesh,
    compiler_params=pltpu.CompilerParams(
        needs_layout_passes=False, use_tc_tiling_on_sc=False),
    scratch_types=[
        pltpu.VMEM((4, CHUNK), jnp.int32),
        pltpu.VMEM((4, CHUNK), jnp.int32),
        pltpu.VMEM((2, CHUNK, 16), jnp.float32),
        pltpu.VMEM((2, CHUNK, HA), jnp.float32),
        pltpu.VMEM((16,), jnp.float32),
        pltpu.VMEM((16,), jnp.float32),
        pltpu.VMEM_SHARED((N, HA), jnp.float32),
        pltpu.SemaphoreType.DMA((2,)),
        pltpu.SemaphoreType.DMA((2,)),
        pltpu.SemaphoreType.DMA((2,)),
    ],
)
def _sc_main(h_hbm, sa_hbm, src_hbm, dst_hbm, ma_hbm, mb_hbm, z_hbm,
             acc_hbm,
             src_v, dst_v, sd_v, rows_v, ma_v, mb_v, acc_sh,
             gsem, ssem, isem):
    cid = lax.axis_index("c")
    tid = lax.axis_index("s")
    wid = cid * NS + tid
    row0 = wid * NCHP

    pltpu.sync_copy(ma_hbm.at[pl.ds(0, 16)], ma_v)
    pltpu.sync_copy(mb_hbm.at[pl.ds(0, 16)], mb_v)

    @pl.when(tid < NS - 1)
    def _():
        pltpu.sync_copy(z_hbm.at[pl.ds(tid * 632, 632)],
                        acc_sh.at[pl.ds(tid * 632, 632)])

    @pl.when(tid == NS - 1)
    def _():
        pltpu.sync_copy(z_hbm.at[pl.ds(9480, 520)],
                        acc_sh.at[pl.ds(9480, 520)])

    plsc.subcore_barrier()

    m0 = ma_v[...] + mb_v[...]
    m4 = jnp.maximum(m0, 0.01 * m0)
    dmask = jnp.where(lax.iota(jnp.int32, 16) < NH, 1.0, 0.0
                      ).astype(jnp.float32)

    pltpu.sync_copy(src_hbm.at[row0], src_v.at[0])
    pltpu.sync_copy(dst_hbm.at[row0], dst_v.at[0])
    pltpu.sync_copy(src_hbm.at[row0 + 1], src_v.at[1])
    pltpu.sync_copy(dst_hbm.at[row0 + 1], dst_v.at[1])
    pltpu.async_copy(h_hbm.at[src_v.at[0]], rows_v.at[0], gsem.at[0])
    pltpu.async_copy(sa_hbm.at[dst_v.at[0]], sd_v.at[0], gsem.at[0])

    def chunk_body(c, carry):
        b = lax.rem(c, 2)
        b1 = lax.rem(c + 1, 2)
        sl = lax.rem(c, 4)
        sl1 = lax.rem(c + 1, 4)
        sl2 = lax.rem(c + 2, 4)

        pltpu.make_async_copy(
            h_hbm.at[src_v.at[sl]], rows_v.at[b], gsem.at[b]).wait()
        pltpu.make_async_copy(
            sa_hbm.at[dst_v.at[sl]], sd_v.at[b], gsem.at[b]).wait()

        @pl.when(jnp.logical_and(c >= 1, c + 1 < NCH))
        def _():
            pltpu.make_async_copy(
                src_hbm.at[row0 + c + 1], src_v.at[sl1], isem.at[b1]).wait()
            pltpu.make_async_copy(
                dst_hbm.at[row0 + c + 1], dst_v.at[sl1], isem.at[b1]).wait()

        @pl.when(c >= 1)
        def _():
            pltpu.make_async_copy(
                rows_v.at[b1], acc_sh.at[dst_v.at[sl1]], ssem.at[b1]).wait()

        @pl.when(c + 1 < NCH)
        def _():
            pltpu.async_copy(
                h_hbm.at[src_v.at[sl1]], rows_v.at[b1], gsem.at[b1])
            pltpu.async_copy(
                sa_hbm.at[dst_v.at[sl1]], sd_v.at[b1], gsem.at[b1])

        @pl.when(c + 2 < NCH)
        def _():
            pltpu.async_copy(
                src_hbm.at[row0 + c + 2], src_v.at[sl2], isem.at[b])
            pltpu.async_copy(
                dst_hbm.at[row0 + c + 2], dst_v.at[sl2], isem.at[b])

        def quad_body(q, qcarry):
            for k in range(4):
                e = q * 4 + k
                z = sd_v[b, e, pl.ds(0, 16)] + rows_v[b, e, pl.ds(HF, 16)]
                l = jnp.maximum(z, 0.01 * z)
                p16 = jnp.exp(l - m4)
                for hd in range(NH):
                    cv = jnp.full((16,), p16[hd], jnp.float32)
                    lo = rows_v[b, e, pl.ds(hd * 32, 16)] * cv
                    rows_v[b, e, pl.ds(hd * 32, 16)] = lo
                    hi = rows_v[b, e, pl.ds(hd * 32 + 16, 16)] * cv
                    rows_v[b, e, pl.ds(hd * 32 + 16, 16)] = hi
                rows_v[b, e, pl.ds(HF, 16)] = p16 * dmask
            return qcarry

        lax.fori_loop(0, CHUNK // 4, quad_body, 0)

        pltpu.async_copy(
            rows_v.at[b], acc_sh.at[dst_v.at[sl]], ssem.at[b], add=True)
        return carry

    lax.fori_loop(0, NCH, chunk_body, 0)
    pltpu.make_async_copy(
        rows_v.at[(NCH - 1) % 2],
        acc_sh.at[dst_v.at[(NCH - 1) % 4]],
        ssem.at[(NCH - 1) % 2]).wait()

    plsc.subcore_barrier()

    @pl.when(tid < NS - 1)
    def _():
        pltpu.sync_copy(acc_sh.at[pl.ds(tid * 632, 632)],
                        acc_hbm.at[pl.ds(cid * N + tid * 632, 632)])

    @pl.when(tid == NS - 1)
    def _():
        pltpu.sync_copy(acc_sh.at[pl.ds(9480, 520)],
                        acc_hbm.at[pl.ds(cid * N + 9480, 520)])


def _tc_combine(a0_ref, a1_ref, b_ref, o_ref):
    a = a0_ref[...] + a1_ref[...]
    d = a[:, HF:HF + NH]
    r = 1.0 / (d + 1e-16)
    o_ref[...] = a[:, :HF] * jnp.dot(
        r, b_ref[...], preferred_element_type=jnp.float32)


def _run_tc_combine(acc, bmat):
    nb = N // ROWS_BLK
    return pl.pallas_call(
        _tc_combine,
        grid=(nb,),
        in_specs=[
            pl.BlockSpec((ROWS_BLK, HA), lambda i: (i, 0)),
            pl.BlockSpec((ROWS_BLK, HA), lambda i: (nb + i, 0)),
            pl.BlockSpec((NH, HF), lambda i: (0, 0)),
        ],
        out_specs=pl.BlockSpec((ROWS_BLK, HF), lambda i: (i, 0)),
        out_shape=jax.ShapeDtypeStruct((N, HF), jnp.float32),
    )(acc, acc, bmat)


def kernel(x, edge_index, counts, out_edge_idx, layer_i, W, att_i, att_j):
    ei = edge_index.astype(jnp.int32).reshape(2, NW, NCH, CHUNK)
    ei = jnp.pad(ei, ((0, 0), (0, 0), (0, NCHP - NCH), (0, 0)))
    src = ei[0].reshape(NW * NCHP, CHUNK)
    dst = ei[1].reshape(NW * NCHP, CHUNK)
    wt = W.T.astype(jnp.float32)
    bmat = jnp.repeat(jnp.eye(NH, dtype=jnp.float32), HF // NH, axis=1)
    pad12 = jnp.zeros((HF, 12), jnp.float32)
    amat_i = jnp.concatenate([bmat.T * att_i.reshape(-1)[:, None], pad12], 1)
    amat_j = jnp.concatenate([bmat.T * att_j.reshape(-1)[:, None], pad12], 1)
    h, sa, ma, mb = _run_tc_front(x, wt, amat_i, amat_j)
    z = jnp.zeros((N, HA), jnp.float32)
    acc = _sc_main(h, sa, src, dst, ma.reshape(-1), mb.reshape(-1), z)
    return _run_tc_combine(acc, bmat)
